# baseline (speedup 1.0000x reference)
import functools

import jax
import jax.numpy as jnp
from jax import lax
from jax.experimental import pallas as pl
from jax.experimental.pallas import tpu as pltpu
from jax.experimental.pallas import tpu_sc as plsc

BATCH = 16384
FIELDS = 26
EMBED_DIM = 32

_B = BATCH * FIELDS
_NW = 32
_CHUNK = 1024
_NBG = BATCH // _CHUNK
_NBLK = FIELDS * _NBG
_BLK_PER_W = _NBLK // _NW
_STG = _CHUNK * EMBED_DIM
_PLANE = _STG // 4
_OUT_WORDS = FIELDS * EMBED_DIM * BATCH

_mesh = plsc.VectorSubcoreMesh(core_axis_name="c", subcore_axis_name="s")


@functools.partial(
    pl.kernel,
    mesh=_mesh,
    compiler_params=pltpu.CompilerParams(
        use_tc_tiling_on_sc=False, needs_layout_passes=False),
    out_type=jax.ShapeDtypeStruct((_OUT_WORDS,), jnp.float32),
    scratch_types=[
        tuple(pltpu.VMEM((_CHUNK,), jnp.int32) for _ in range(2)),
        tuple(pltpu.VMEM((_CHUNK, EMBED_DIM), jnp.float32) for _ in range(2)),
        pltpu.VMEM((_STG,), jnp.float32),
        tuple(pltpu.SemaphoreType.DMA for _ in range(2)),
        pltpu.SemaphoreType.DMA,
    ],
)
def _gather_all(idx_hbm, table_hbm, out_hbm,
                idx_v, gbuf, stage, gsem, wsem):
    wid = lax.axis_index("s") * 2 + lax.axis_index("c")
    blk0 = wid * _BLK_PER_W

    iota = jax.lax.iota(jnp.int32, 16)

    def idx_off(beta):
        f = beta >> 4
        bg = beta & 15
        return f * BATCH + bg * _CHUNK

    def start_gather(beta, p):
        pltpu.sync_copy(idx_hbm.at[pl.ds(idx_off(beta), _CHUNK)], idx_v[p])
        pltpu.async_copy(table_hbm.at[idx_v[p]], gbuf[p], gsem[p])

    def wait_gather(p):
        pltpu.make_async_copy(
            table_hbm.at[idx_v[p]], gbuf[p], gsem[p]).wait()

    def block_body(beta, p):
        g = gbuf[p]

        @plsc.parallel_loop(0, EMBED_DIM, unroll=4)
        def _perm(d):
            dbase = (d >> 3) * _PLANE + (d & 7) * 128
            cvec = d + jnp.zeros((16,), jnp.int32)
            for j in range(_CHUNK // 16):
                rvec = j * 16 + iota
                v = plsc.load_gather(g, [rvec, cvec])
                off = (j >> 3) * 1024 + (j & 7) * 16
                stage[pl.ds(dbase + off, 16)] = v

        f = beta >> 4
        bg = beta & 15
        for dt in range(4):
            dst = f * 524288 + dt * 131072 + bg * _PLANE
            pltpu.async_copy(
                stage.at[pl.ds(dt * _PLANE, _PLANE)],
                out_hbm.at[pl.ds(dst, _PLANE)],
                wsem)

    def wait_writes():
        for dt in range(4):
            pltpu.make_async_copy(
                stage.at[pl.ds(dt * _PLANE, _PLANE)],
                out_hbm.at[pl.ds(dt * _PLANE, _PLANE)],
                wsem).wait()

    start_gather(blk0, 0)

    @pl.loop(0, _BLK_PER_W)
    def _blocks(t):
        even = lax.rem(t, 2) == 0

        @pl.when(t + 1 < _BLK_PER_W)
        def _():
            @pl.when(even)
            def _():
                start_gather(blk0 + t + 1, 1)

            @pl.when(jnp.logical_not(even))
            def _():
                start_gather(blk0 + t + 1, 0)

        @pl.when(t >= 1)
        def _():
            wait_writes()

        @pl.when(even)
        def _():
            wait_gather(0)
            block_body(blk0 + t, 0)

        @pl.when(jnp.logical_not(even))
        def _():
            wait_gather(1)
            block_body(blk0 + t, 1)

    wait_writes()


def kernel(cat_idx, table):
    idx_flat = cat_idx.T.reshape(_B).astype(jnp.int32)
    out = _gather_all(idx_flat, table)
    out6 = out.reshape(FIELDS, 4, 128, 8, 128)
    return out6.transpose(2, 4, 0, 1, 3).reshape(BATCH, FIELDS, EMBED_DIM)


# speedup vs baseline: 1.2603x; 1.2603x over previous
import functools

import jax
import jax.numpy as jnp
from jax import lax
from jax.experimental import pallas as pl
from jax.experimental.pallas import tpu as pltpu
from jax.experimental.pallas import tpu_sc as plsc

BATCH = 16384
FIELDS = 26
EMBED_DIM = 32

_B = BATCH * FIELDS
_NW = 32
_CHUNK = 1024
_NBG = BATCH // _CHUNK
_NBLK = FIELDS * _NBG
_BLK_PER_W = _NBLK // _NW
_STG = _CHUNK * EMBED_DIM
_PLANE = _STG // 4
_OUT_WORDS = FIELDS * EMBED_DIM * BATCH

_mesh = plsc.VectorSubcoreMesh(core_axis_name="c", subcore_axis_name="s")


@functools.partial(
    pl.kernel,
    mesh=_mesh,
    compiler_params=pltpu.CompilerParams(
        use_tc_tiling_on_sc=False, needs_layout_passes=False),
    out_type=jax.ShapeDtypeStruct((_OUT_WORDS,), jnp.float32),
    scratch_types=[
        tuple(pltpu.VMEM((_CHUNK,), jnp.int32) for _ in range(2)),
        tuple(pltpu.VMEM((_CHUNK, EMBED_DIM), jnp.float32) for _ in range(2)),
        pltpu.VMEM((_STG,), jnp.float32),
        tuple(pltpu.SemaphoreType.DMA for _ in range(2)),
        pltpu.SemaphoreType.DMA,
    ],
)
def _gather_all(idx_hbm, table_hbm, out_hbm,
                idx_v, gbuf, stage, gsem, wsem):
    wid = lax.axis_index("s") * 2 + lax.axis_index("c")
    blk0 = wid * _BLK_PER_W

    iota = jax.lax.iota(jnp.int32, 16)

    def idx_off(beta):
        f = beta >> 4
        bg = beta & 15
        return f * BATCH + bg * _CHUNK

    def start_gather(beta, p):
        pltpu.sync_copy(idx_hbm.at[pl.ds(idx_off(beta), _CHUNK)], idx_v[p])
        pltpu.async_copy(table_hbm.at[idx_v[p]], gbuf[p], gsem[p])

    def wait_gather(p):
        pltpu.make_async_copy(
            table_hbm.at[idx_v[p]], gbuf[p], gsem[p]).wait()

    dvecs = [(c + iota) & 31 for c in range(EMBED_DIM)]
    dstparts = [(dv >> 3) * _PLANE + (dv & 7) * 128 + iota for dv in dvecs]

    def block_body(beta, p):
        g = gbuf[p]

        @plsc.parallel_loop(0, _CHUNK // 16, unroll=2)
        def _perm(j):
            rvec = j * 16 + iota
            sbase = (j >> 3) * 1024 + (j & 7) * 16
            for c in range(EMBED_DIM):
                v = plsc.load_gather(g, [rvec, dvecs[c]])
                plsc.store_scatter(stage, [dstparts[c] + sbase], v)

        f = beta >> 4
        bg = beta & 15
        for dt in range(4):
            dst = f * 524288 + dt * 131072 + bg * _PLANE
            pltpu.async_copy(
                stage.at[pl.ds(dt * _PLANE, _PLANE)],
                out_hbm.at[pl.ds(dst, _PLANE)],
                wsem)

    def wait_writes():
        for dt in range(4):
            pltpu.make_async_copy(
                stage.at[pl.ds(dt * _PLANE, _PLANE)],
                out_hbm.at[pl.ds(dt * _PLANE, _PLANE)],
                wsem).wait()

    start_gather(blk0, 0)

    @pl.loop(0, _BLK_PER_W)
    def _blocks(t):
        even = lax.rem(t, 2) == 0

        @pl.when(t + 1 < _BLK_PER_W)
        def _():
            @pl.when(even)
            def _():
                start_gather(blk0 + t + 1, 1)

            @pl.when(jnp.logical_not(even))
            def _():
                start_gather(blk0 + t + 1, 0)

        @pl.when(t >= 1)
        def _():
            wait_writes()

        @pl.when(even)
        def _():
            wait_gather(0)
            block_body(blk0 + t, 0)

        @pl.when(jnp.logical_not(even))
        def _():
            wait_gather(1)
            block_body(blk0 + t, 1)

    wait_writes()


def kernel(cat_idx, table):
    idx_flat = cat_idx.T.reshape(_B).astype(jnp.int32)
    out = _gather_all(idx_flat, table)
    out6 = out.reshape(FIELDS, 4, 128, 8, 128)
    return out6.transpose(2, 4, 0, 1, 3).reshape(BATCH, FIELDS, EMBED_DIM)


# speedup vs baseline: 3.6571x; 2.9019x over previous
import functools

import jax
import jax.numpy as jnp
from jax import lax
from jax.experimental import pallas as pl
from jax.experimental.pallas import tpu as pltpu
from jax.experimental.pallas import tpu_sc as plsc

BATCH = 16384
FIELDS = 26
EMBED_DIM = 32
NUM_ROWS = 1000000

_B = BATCH * FIELDS
_NW = 32
_CHUNK = 1024
_NBG = BATCH // _CHUNK
_NBLK = FIELDS * _NBG
_BLK_PER_W = _NBLK // _NW
_STG = _CHUNK * EMBED_DIM
_PLANE = _STG // 4
_OUT_WORDS = FIELDS * EMBED_DIM * BATCH

_mesh = plsc.VectorSubcoreMesh(core_axis_name="c", subcore_axis_name="s")


_NTC = 7813
_TC_PER_W = 244
_CT = 4
_STEPS = _TC_PER_W // _CT


@functools.partial(
    pl.kernel,
    mesh=_mesh,
    compiler_params=pltpu.CompilerParams(
        use_tc_tiling_on_sc=True, needs_layout_passes=False),
    out_type=jax.ShapeDtypeStruct((NUM_ROWS * EMBED_DIM,), jnp.float32),
    scratch_types=[
        tuple(pltpu.VMEM((EMBED_DIM, _CT * 128), jnp.float32) for _ in range(2)),
        tuple(pltpu.VMEM((_CT * 128 * EMBED_DIM,), jnp.float32) for _ in range(2)),
        pltpu.VMEM((EMBED_DIM, 128), jnp.float32),
        pltpu.VMEM((128 * EMBED_DIM,), jnp.float32),
        tuple(pltpu.SemaphoreType.DMA for _ in range(2)),
        tuple(pltpu.SemaphoreType.DMA for _ in range(2)),
        pltpu.SemaphoreType.DMA,
    ],
)
def _detile_all(tab_hbm, lin_hbm, vbuf, stage, vbufx, stagex, isem, osem, xsem):
    wid = lax.axis_index("s") * 2 + lax.axis_index("c")
    base = wid * _TC_PER_W

    iota = jax.lax.iota(jnp.int32, 16)
    dvecs = [(d0 + iota) & 31 for d0 in range(EMBED_DIM)]

    def col0(s):
        return (base + s * _CT) * 128

    def start_in(s, p):
        pltpu.async_copy(
            tab_hbm.at[:, pl.ds(col0(s), _CT * 128)], vbuf[p], isem[p])

    def wait_in(p):
        pltpu.make_async_copy(
            tab_hbm.at[:, pl.ds(0, _CT * 128)], vbuf[p], isem[p]).wait()

    def start_out(s, p):
        pltpu.async_copy(
            stage[p], lin_hbm.at[pl.ds(col0(s) * EMBED_DIM, _CT * 128 * EMBED_DIM)],
            osem[p])

    def wait_out(p):
        pltpu.make_async_copy(
            stage[p], lin_hbm.at[pl.ds(0, _CT * 128 * EMBED_DIM)], osem[p]).wait()

    def transpose_step(p):
        v = vbuf[p]
        st = stage[p]

        @plsc.parallel_loop(0, _CT * 128 // 16, unroll=2)
        def _tr(j):
            ccvec = j * 16 + iota
            ccv32 = ccvec * EMBED_DIM
            for d0 in range(EMBED_DIM):
                x = plsc.load_gather(v, [dvecs[d0], ccvec])
                plsc.store_scatter(st, [ccv32 + dvecs[d0]], x)

    start_in(0, 0)

    @pl.loop(0, _STEPS)
    def _steps(s):
        even = lax.rem(s, 2) == 0

        @pl.when(s + 1 < _STEPS)
        def _():
            @pl.when(even)
            def _():
                start_in(s + 1, 1)

            @pl.when(jnp.logical_not(even))
            def _():
                start_in(s + 1, 0)

        @pl.when(even)
        def _():
            wait_in(0)

            @pl.when(s >= 2)
            def _():
                wait_out(0)
            transpose_step(0)
            start_out(s, 0)

        @pl.when(jnp.logical_not(even))
        def _():
            wait_in(1)

            @pl.when(s >= 2)
            def _():
                wait_out(1)
            transpose_step(1)
            start_out(s, 1)

    wait_out(0)
    wait_out(1)

    @pl.when(wid < 5)
    def _():
        ct = _TC_PER_W * _NW + wid
        pltpu.sync_copy(tab_hbm.at[:, pl.ds(ct * 128, 128)], vbufx)

        @plsc.parallel_loop(0, 8, unroll=2)
        def _trx(j):
            ccvec = j * 16 + iota
            ccv32 = ccvec * EMBED_DIM
            for d0 in range(EMBED_DIM):
                x = plsc.load_gather(vbufx, [dvecs[d0], ccvec])
                plsc.store_scatter(stagex, [ccv32 + dvecs[d0]], x)

        @pl.when(wid < 4)
        def _():
            pltpu.make_async_copy(
                stagex, lin_hbm.at[pl.ds(ct * 4096, 4096)], xsem).start()
            pltpu.make_async_copy(
                stagex, lin_hbm.at[pl.ds(ct * 4096, 4096)], xsem).wait()

        @pl.when(wid == 4)
        def _():
            pltpu.make_async_copy(
                stagex.at[pl.ds(0, 2048)],
                lin_hbm.at[pl.ds(ct * 4096, 2048)], xsem).start()
            pltpu.make_async_copy(
                stagex.at[pl.ds(0, 2048)],
                lin_hbm.at[pl.ds(ct * 4096, 2048)], xsem).wait()


@functools.partial(
    pl.kernel,
    mesh=_mesh,
    compiler_params=pltpu.CompilerParams(
        use_tc_tiling_on_sc=False, needs_layout_passes=False),
    out_type=jax.ShapeDtypeStruct((_OUT_WORDS,), jnp.float32),
    scratch_types=[
        tuple(pltpu.VMEM((_CHUNK,), jnp.int32) for _ in range(2)),
        tuple(pltpu.VMEM((_CHUNK, EMBED_DIM), jnp.float32) for _ in range(2)),
        pltpu.VMEM((_STG,), jnp.float32),
        tuple(pltpu.SemaphoreType.DMA for _ in range(2)),
        pltpu.SemaphoreType.DMA,
    ],
)
def _gather_all(idx_hbm, table_hbm, out_hbm,
                idx_v, gbuf, stage, gsem, wsem):
    wid = lax.axis_index("s") * 2 + lax.axis_index("c")
    blk0 = wid * _BLK_PER_W

    iota = jax.lax.iota(jnp.int32, 16)

    def idx_off(beta):
        f = beta >> 4
        bg = beta & 15
        return f * BATCH + bg * _CHUNK

    def start_gather(beta, p):
        pltpu.sync_copy(idx_hbm.at[pl.ds(idx_off(beta), _CHUNK)], idx_v[p])
        pltpu.async_copy(table_hbm.at[idx_v[p]], gbuf[p], gsem[p])

    def wait_gather(p):
        pltpu.make_async_copy(
            table_hbm.at[idx_v[p]], gbuf[p], gsem[p]).wait()

    dvecs = [(c + iota) & 31 for c in range(EMBED_DIM)]
    dstparts = [(dv >> 3) * _PLANE + (dv & 7) * 128 + iota for dv in dvecs]

    def block_body(beta, p):
        g = gbuf[p]

        @plsc.parallel_loop(0, _CHUNK // 16, unroll=2)
        def _perm(j):
            rvec = j * 16 + iota
            sbase = (j >> 3) * 1024 + (j & 7) * 16
            for c in range(EMBED_DIM):
                v = plsc.load_gather(g, [rvec, dvecs[c]])
                plsc.store_scatter(stage, [dstparts[c] + sbase], v)

        f = beta >> 4
        bg = beta & 15
        for dt in range(4):
            dst = f * 524288 + dt * 131072 + bg * _PLANE
            pltpu.async_copy(
                stage.at[pl.ds(dt * _PLANE, _PLANE)],
                out_hbm.at[pl.ds(dst, _PLANE)],
                wsem)

    def wait_writes():
        for dt in range(4):
            pltpu.make_async_copy(
                stage.at[pl.ds(dt * _PLANE, _PLANE)],
                out_hbm.at[pl.ds(dt * _PLANE, _PLANE)],
                wsem).wait()

    start_gather(blk0, 0)

    @pl.loop(0, _BLK_PER_W)
    def _blocks(t):
        even = lax.rem(t, 2) == 0

        @pl.when(t + 1 < _BLK_PER_W)
        def _():
            @pl.when(even)
            def _():
                start_gather(blk0 + t + 1, 1)

            @pl.when(jnp.logical_not(even))
            def _():
                start_gather(blk0 + t + 1, 0)

        @pl.when(t >= 1)
        def _():
            wait_writes()

        @pl.when(even)
        def _():
            wait_gather(0)
            block_body(blk0 + t, 0)

        @pl.when(jnp.logical_not(even))
        def _():
            wait_gather(1)
            block_body(blk0 + t, 1)

    wait_writes()


def kernel(cat_idx, table):
    idx_flat = cat_idx.T.reshape(_B).astype(jnp.int32)
    table_lin = _detile_all(table.T).reshape(NUM_ROWS, EMBED_DIM)
    out = _gather_all(idx_flat, table_lin)
    out6 = out.reshape(FIELDS, 4, 128, 8, 128)
    return out6.transpose(2, 4, 0, 1, 3).reshape(BATCH, FIELDS, EMBED_DIM)


# speedup vs baseline: 4.1926x; 1.1464x over previous
import functools

import jax
import jax.numpy as jnp
from jax import lax
from jax.experimental import pallas as pl
from jax.experimental.pallas import tpu as pltpu
from jax.experimental.pallas import tpu_sc as plsc

BATCH = 16384
FIELDS = 26
EMBED_DIM = 32
NUM_ROWS = 1000000

_B = BATCH * FIELDS
_NW = 32
_CHUNK = 1024
_NBG = BATCH // _CHUNK
_NBLK = FIELDS * _NBG
_BLK_PER_W = _NBLK // _NW
_STG = _CHUNK * EMBED_DIM
_PLANE = _STG // 4
_OUT_WORDS = FIELDS * EMBED_DIM * BATCH

_mesh = plsc.VectorSubcoreMesh(core_axis_name="c", subcore_axis_name="s")


_NTC = 7813
_TC_PER_W = 244
_CT = 4
_STEPS = _TC_PER_W // _CT


@functools.partial(
    pl.kernel,
    mesh=_mesh,
    compiler_params=pltpu.CompilerParams(
        use_tc_tiling_on_sc=True, needs_layout_passes=False),
    out_type=jax.ShapeDtypeStruct((NUM_ROWS * EMBED_DIM,), jnp.float32),
    scratch_types=[
        tuple(pltpu.VMEM((EMBED_DIM, _CT * 128), jnp.float32) for _ in range(2)),
        tuple(pltpu.VMEM((_CT * 128 * EMBED_DIM,), jnp.float32) for _ in range(2)),
        pltpu.VMEM((EMBED_DIM, 128), jnp.float32),
        pltpu.VMEM((128 * EMBED_DIM,), jnp.float32),
        tuple(pltpu.SemaphoreType.DMA for _ in range(2)),
        tuple(pltpu.SemaphoreType.DMA for _ in range(2)),
        pltpu.SemaphoreType.DMA,
    ],
)
def _detile_all(tab_hbm, lin_hbm, vbuf, stage, vbufx, stagex, isem, osem, xsem):
    wid = lax.axis_index("s") * 2 + lax.axis_index("c")
    base = wid * _TC_PER_W

    iota = jax.lax.iota(jnp.int32, 16)
    dvecs = [(d0 + iota) & 31 for d0 in range(EMBED_DIM)]

    def col0(s):
        return (base + s * _CT) * 128

    def start_in(s, p):
        pltpu.async_copy(
            tab_hbm.at[:, pl.ds(col0(s), _CT * 128)], vbuf[p], isem[p])

    def wait_in(p):
        pltpu.make_async_copy(
            tab_hbm.at[:, pl.ds(0, _CT * 128)], vbuf[p], isem[p]).wait()

    def start_out(s, p):
        pltpu.async_copy(
            stage[p], lin_hbm.at[pl.ds(col0(s) * EMBED_DIM, _CT * 128 * EMBED_DIM)],
            osem[p])

    def wait_out(p):
        pltpu.make_async_copy(
            stage[p], lin_hbm.at[pl.ds(0, _CT * 128 * EMBED_DIM)], osem[p]).wait()

    def transpose_step(p):
        v = vbuf[p]
        st = stage[p]

        @plsc.parallel_loop(0, _CT * 128 // 16, unroll=4)
        def _tr(j):
            ccvec = j * 16 + iota
            ccv32 = ccvec * EMBED_DIM
            for d0 in range(EMBED_DIM):
                x = plsc.load_gather(v, [dvecs[d0], ccvec])
                plsc.store_scatter(st, [ccv32 + dvecs[d0]], x)

    start_in(0, 0)

    @pl.loop(0, _STEPS)
    def _steps(s):
        even = lax.rem(s, 2) == 0

        @pl.when(s + 1 < _STEPS)
        def _():
            @pl.when(even)
            def _():
                start_in(s + 1, 1)

            @pl.when(jnp.logical_not(even))
            def _():
                start_in(s + 1, 0)

        @pl.when(even)
        def _():
            wait_in(0)

            @pl.when(s >= 2)
            def _():
                wait_out(0)
            transpose_step(0)
            start_out(s, 0)

        @pl.when(jnp.logical_not(even))
        def _():
            wait_in(1)

            @pl.when(s >= 2)
            def _():
                wait_out(1)
            transpose_step(1)
            start_out(s, 1)

    wait_out(0)
    wait_out(1)

    @pl.when(wid < 5)
    def _():
        ct = _TC_PER_W * _NW + wid
        pltpu.sync_copy(tab_hbm.at[:, pl.ds(ct * 128, 128)], vbufx)

        @plsc.parallel_loop(0, 8, unroll=4)
        def _trx(j):
            ccvec = j * 16 + iota
            ccv32 = ccvec * EMBED_DIM
            for d0 in range(EMBED_DIM):
                x = plsc.load_gather(vbufx, [dvecs[d0], ccvec])
                plsc.store_scatter(stagex, [ccv32 + dvecs[d0]], x)

        @pl.when(wid < 4)
        def _():
            pltpu.make_async_copy(
                stagex, lin_hbm.at[pl.ds(ct * 4096, 4096)], xsem).start()
            pltpu.make_async_copy(
                stagex, lin_hbm.at[pl.ds(ct * 4096, 4096)], xsem).wait()

        @pl.when(wid == 4)
        def _():
            pltpu.make_async_copy(
                stagex.at[pl.ds(0, 2048)],
                lin_hbm.at[pl.ds(ct * 4096, 2048)], xsem).start()
            pltpu.make_async_copy(
                stagex.at[pl.ds(0, 2048)],
                lin_hbm.at[pl.ds(ct * 4096, 2048)], xsem).wait()


@functools.partial(
    pl.kernel,
    mesh=_mesh,
    compiler_params=pltpu.CompilerParams(
        use_tc_tiling_on_sc=False, needs_layout_passes=False),
    out_type=jax.ShapeDtypeStruct((_OUT_WORDS,), jnp.float32),
    scratch_types=[
        tuple(pltpu.VMEM((_CHUNK,), jnp.int32) for _ in range(2)),
        tuple(pltpu.VMEM((_CHUNK, EMBED_DIM), jnp.float32) for _ in range(2)),
        pltpu.VMEM((_STG,), jnp.float32),
        tuple(pltpu.SemaphoreType.DMA for _ in range(2)),
        pltpu.SemaphoreType.DMA,
    ],
)
def _gather_all(idx_hbm, table_hbm, out_hbm,
                idx_v, gbuf, stage, gsem, wsem):
    wid = lax.axis_index("s") * 2 + lax.axis_index("c")
    blk0 = wid * _BLK_PER_W

    iota = jax.lax.iota(jnp.int32, 16)

    def idx_off(beta):
        f = beta >> 4
        bg = beta & 15
        return f * BATCH + bg * _CHUNK

    def start_gather(beta, p):
        pltpu.sync_copy(idx_hbm.at[pl.ds(idx_off(beta), _CHUNK)], idx_v[p])
        pltpu.async_copy(table_hbm.at[idx_v[p]], gbuf[p], gsem[p])

    def wait_gather(p):
        pltpu.make_async_copy(
            table_hbm.at[idx_v[p]], gbuf[p], gsem[p]).wait()

    dvecs = [(c + iota) & 31 for c in range(EMBED_DIM)]
    dstparts = [(dv >> 3) * _PLANE + (dv & 7) * 128 + iota for dv in dvecs]

    def block_body(beta, p):
        g = gbuf[p]

        @plsc.parallel_loop(0, _CHUNK // 16, unroll=4)
        def _perm(j):
            rvec = j * 16 + iota
            sbase = (j >> 3) * 1024 + (j & 7) * 16
            for c in range(EMBED_DIM):
                v = plsc.load_gather(g, [rvec, dvecs[c]])
                plsc.store_scatter(stage, [dstparts[c] + sbase], v)

        f = beta >> 4
        bg = beta & 15
        for dt in range(4):
            dst = f * 524288 + dt * 131072 + bg * _PLANE
            pltpu.async_copy(
                stage.at[pl.ds(dt * _PLANE, _PLANE)],
                out_hbm.at[pl.ds(dst, _PLANE)],
                wsem)

    def wait_writes():
        for dt in range(4):
            pltpu.make_async_copy(
                stage.at[pl.ds(dt * _PLANE, _PLANE)],
                out_hbm.at[pl.ds(dt * _PLANE, _PLANE)],
                wsem).wait()

    start_gather(blk0, 0)

    @pl.loop(0, _BLK_PER_W)
    def _blocks(t):
        even = lax.rem(t, 2) == 0

        @pl.when(t + 1 < _BLK_PER_W)
        def _():
            @pl.when(even)
            def _():
                start_gather(blk0 + t + 1, 1)

            @pl.when(jnp.logical_not(even))
            def _():
                start_gather(blk0 + t + 1, 0)

        @pl.when(t >= 1)
        def _():
            wait_writes()

        @pl.when(even)
        def _():
            wait_gather(0)
            block_body(blk0 + t, 0)

        @pl.when(jnp.logical_not(even))
        def _():
            wait_gather(1)
            block_body(blk0 + t, 1)

    wait_writes()


def kernel(cat_idx, table):
    idx_flat = cat_idx.T.reshape(_B).astype(jnp.int32)
    table_lin = _detile_all(table.T).reshape(NUM_ROWS, EMBED_DIM)
    out = _gather_all(idx_flat, table_lin)
    out6 = out.reshape(FIELDS, 4, 128, 8, 128)
    return out6.transpose(2, 4, 0, 1, 3).reshape(BATCH, FIELDS, EMBED_DIM)
